# Initial kernel scaffold; baseline (speedup 1.0000x reference)
import jax
import jax.numpy as jnp
from jax.experimental import pallas as pl


def kernel(x, global_node_idx, edge_index, W_lin, b_lin, W_conv, b_conv, W_attn_l, b_attn_l, W_attn_r, b_attn_r, alpha_weights):
    raise NotImplementedError("write your pallas kernel here")


# speedup vs baseline: 24.2121x; 24.2121x over previous
import functools

import jax
import jax.numpy as jnp
from jax import lax
from jax.experimental import pallas as pl
from jax.experimental.pallas import tpu as pltpu
from jax.experimental.pallas import tpu_sc as plsc

N = 10000
D = 128
E = 320000
GW = 144
NT = 32
NSUB = 16
CH = 128
KCH = (E + NT * CH - 1) // (NT * CH)
EP = NT * CH * KCH
NPAD = N + 112
SLICE = NPAD // NSUB
BLK = 1000


def _prep_body(x_ref, wlin_ref, blin_ref, war_ref, scal_ref, h_ref, g_ref):
    x = x_ref[...]
    h = lax.dot_general(x, wlin_ref[...], (((1,), (1,)), ((), ())),
                        preferred_element_type=jnp.float32) + blin_ref[...]
    h_ref[...] = h
    ar = jnp.sum(h * war_ref[...], axis=1, keepdims=True) + scal_ref[0]
    u = jnp.exp(scal_ref[1] * ar)
    g_ref[...] = jnp.concatenate(
        [u * h, jnp.broadcast_to(u, (BLK, GW - D))], axis=1)


def _prep(x, W_lin, b_lin, W_attn_r, scal):
    return pl.pallas_call(
        _prep_body,
        grid=(N // BLK,),
        in_specs=[
            pl.BlockSpec((BLK, D), lambda i: (i, 0)),
            pl.BlockSpec((D, D), lambda i: (0, 0)),
            pl.BlockSpec((1, D), lambda i: (0, 0)),
            pl.BlockSpec((1, D), lambda i: (0, 0)),
            pl.BlockSpec(memory_space=pltpu.SMEM),
        ],
        out_specs=[
            pl.BlockSpec((BLK, D), lambda i: (i, 0)),
            pl.BlockSpec((BLK, GW), lambda i: (i, 0)),
        ],
        out_shape=[
            jax.ShapeDtypeStruct((N, D), jnp.float32),
            jax.ShapeDtypeStruct((N, GW), jnp.float32),
        ],
    )(x, W_lin, b_lin, W_attn_r, scal)


def _sc_body(g_hbm, src_hbm, dst_hbm, zeros_hbm, out_hbm,
             src_v, dst_v, rows_v, acc_sh):
    c = lax.axis_index("c")
    s = lax.axis_index("s")
    wid = c * NSUB + s

    row0 = pl.multiple_of(s * SLICE, 8)
    pltpu.sync_copy(zeros_hbm, acc_sh.at[pl.ds(row0, SLICE)])
    pltpu.sync_copy(src_hbm.at[wid], src_v)
    pltpu.sync_copy(dst_hbm.at[wid], dst_v)
    plsc.subcore_barrier()

    def body(k, carry):
        pltpu.sync_copy(g_hbm.at[src_v.at[k]], rows_v)
        pltpu.sync_copy(rows_v, acc_sh.at[dst_v.at[k]], add=True)
        return carry

    lax.fori_loop(0, KCH, body, 0)

    plsc.subcore_barrier()
    pltpu.sync_copy(acc_sh.at[pl.ds(row0, SLICE)],
                    out_hbm.at[c, pl.ds(row0, SLICE)])


def _sc_aggregate(g, src_p, dst_p, zeros):
    mesh = plsc.VectorSubcoreMesh(core_axis_name="c", subcore_axis_name="s")
    kern = pl.kernel(
        _sc_body,
        out_type=jax.ShapeDtypeStruct((2, NPAD, GW), jnp.float32),
        mesh=mesh,
        scratch_types=[
            pltpu.VMEM((KCH, CH), jnp.int32),
            pltpu.VMEM((KCH, CH), jnp.int32),
            pltpu.VMEM((CH, GW), jnp.float32),
            pltpu.VMEM_SHARED((NPAD, GW), jnp.float32),
        ],
        compiler_params=pltpu.CompilerParams(use_tc_tiling_on_sc=False),
    )
    return kern(g, src_p, dst_p, zeros)


def _combine_body(s_ref, h_ref, x_ref, wc_ref, bc_ref, o_ref):
    ssum = s_ref[0] + s_ref[1]
    agg = ssum[:, :D] / jnp.maximum(ssum[:, D:D + 1], 1e-16)
    wd = wc_ref[0:1, :] - wc_ref[1:2, :]
    dlt = jnp.sum(x_ref[...] * wd, axis=1, keepdims=True) + (bc_ref[0] - bc_ref[1])
    beta0 = 1.0 / (1.0 + jnp.exp(-dlt))
    out = beta0 * agg + (1.0 - beta0) * h_ref[...]
    o_ref[...] = jnp.maximum(out, 0.0)


def _combine(S, h, x, W_conv_pad, b_conv):
    return pl.pallas_call(
        _combine_body,
        grid=(N // BLK,),
        in_specs=[
            pl.BlockSpec((2, BLK, GW), lambda i: (0, i, 0)),
            pl.BlockSpec((BLK, D), lambda i: (i, 0)),
            pl.BlockSpec((BLK, D), lambda i: (i, 0)),
            pl.BlockSpec((8, D), lambda i: (0, 0)),
            pl.BlockSpec(memory_space=pltpu.SMEM),
        ],
        out_specs=pl.BlockSpec((BLK, D), lambda i: (i, 0)),
        out_shape=jax.ShapeDtypeStruct((N, D), jnp.float32),
    )(S, h, x, W_conv_pad, b_conv)


def kernel(x, global_node_idx, edge_index, W_lin, b_lin, W_conv, b_conv,
           W_attn_l, b_attn_l, W_attn_r, b_attn_r, alpha_weights):
    scal = jnp.stack([b_attn_r.astype(jnp.float32).reshape(()),
                      alpha_weights.astype(jnp.float32).reshape(())])
    h, g = _prep(x, W_lin, b_lin.reshape(1, D), W_attn_r.reshape(1, D), scal)

    dst = edge_index[0]
    src = edge_index[1]
    pad = EP - E
    src_p = jnp.concatenate([src, jnp.zeros((pad,), jnp.int32)]).reshape(NT, KCH, CH)
    dst_p = jnp.concatenate([dst, jnp.full((pad,), N, jnp.int32)]).reshape(NT, KCH, CH)
    zeros = jnp.zeros((SLICE, GW), jnp.float32)

    S = _sc_aggregate(g, src_p, dst_p, zeros)

    W_conv_pad = jnp.zeros((8, D), jnp.float32).at[:2].set(W_conv)
    return _combine(S, h, x, W_conv_pad, b_conv)
